# initial kernel scaffold (unmeasured)
import jax
import jax.numpy as jnp
from jax import lax
from jax.experimental import pallas as pl
from jax.experimental.pallas import tpu as pltpu

N = 8
B, S, D = 2, 512, 2048
H, Dh, Dr = 16, 128, 32
HB = 256
RB = 64
F32 = jnp.float32
SCALE = (Dh + Dr) ** -0.5


def _body(x_ref, wdkv_ref, wuk_ref, wuv_ref, wq_ref, wqr_ref, wkr_ref,
          wo_ref, out_ref,
          c_ref, kv_ref, rsbuf_ref, ob_ref, q_ref, qr_ref, kr_ref,
          rs_send_sems, rs_recv_sems, ag_send_sems, ag_recv_sems):
    my = lax.axis_index("i")
    left = (my + N - 1) % N
    right = (my + 1) % N
    own = right

    for b in range(B):
        c_ref[b] = jnp.dot(x_ref[b], wdkv_ref[...], preferred_element_type=F32)
    for j in range(N):
        for b in range(B):
            kv_ref[j, b, :, 0:HB] = jnp.dot(
                c_ref[b], wuk_ref[:, j * HB:(j + 1) * HB],
                preferred_element_type=F32)
            kv_ref[j, b, :, HB:2 * HB] = jnp.dot(
                c_ref[b], wuv_ref[:, j * HB:(j + 1) * HB],
                preferred_element_type=F32)
    for b in range(B):
        q_ref[b] = jnp.dot(x_ref[b], wq_ref[...], preferred_element_type=F32)
        qr_ref[b] = jnp.dot(x_ref[b], wqr_ref[...], preferred_element_type=F32)
        kr_ref[b] = jnp.dot(x_ref[b], wkr_ref[...], preferred_element_type=F32)

    barrier_sem = pltpu.get_barrier_semaphore()
    for nbr in (left, right):
        pl.semaphore_signal(barrier_sem, inc=1, device_id=(nbr,),
                            device_id_type=pl.DeviceIdType.MESH)
    pl.semaphore_wait(barrier_sem, 2)

    for s in range(N - 1):
        sblk = (my - s) % N
        rdma = pltpu.make_async_remote_copy(
            src_ref=kv_ref.at[sblk],
            dst_ref=rsbuf_ref.at[s],
            send_sem=rs_send_sems.at[s],
            recv_sem=rs_recv_sems.at[s],
            device_id=(right,),
            device_id_type=pl.DeviceIdType.MESH,
        )
        rdma.start()
        rdma.wait()
        ablk = (my - s - 1) % N
        for b in range(B):
            kv_ref[ablk, b] = kv_ref[ablk, b] + rsbuf_ref[s, b]

    for b in range(B):
        for h in range(2):
            qh = q_ref[b, :, h * Dh:(h + 1) * Dh]
            kh = kv_ref[own, b, :, h * Dh:(h + 1) * Dh]
            vh = kv_ref[own, b, :, HB + h * Dh:HB + (h + 1) * Dh]
            qrh = qr_ref[b, :, h * Dr:(h + 1) * Dr]
            krb = kr_ref[b]
            sc = lax.dot_general(qh, kh, (((1,), (1,)), ((), ())),
                                 preferred_element_type=F32)
            sc = sc + lax.dot_general(qrh, krb, (((1,), (1,)), ((), ())),
                                      preferred_element_type=F32)
            sc = sc * SCALE
            m = jnp.max(sc, axis=-1, keepdims=True)
            p = jnp.exp(sc - m)
            p = p / jnp.sum(p, axis=-1, keepdims=True)
            ob_ref[own, b, :, h * Dh:(h + 1) * Dh] = jnp.dot(
                p, vh, preferred_element_type=F32)

    for b in range(B):
        out_ref[b] = jnp.dot(ob_ref[own, b], wo_ref[pl.ds(own * HB, HB), :],
                             preferred_element_type=F32)

    for hop in range(N - 1):
        sblk = (my + 1 - hop) % N
        rdma = pltpu.make_async_remote_copy(
            src_ref=ob_ref.at[sblk],
            dst_ref=ob_ref.at[sblk],
            send_sem=ag_send_sems.at[hop],
            recv_sem=ag_recv_sems.at[hop],
            device_id=(right,),
            device_id_type=pl.DeviceIdType.MESH,
        )
        rdma.start()
        rdma.wait()
        rblk = (my - hop) % N
        for b in range(B):
            out_ref[b] = out_ref[b] + jnp.dot(
                ob_ref[rblk, b], wo_ref[pl.ds(rblk * HB, HB), :],
                preferred_element_type=F32)


def kernel(x, Wdkv, Wuk, Wuv, Wq, Wqr, Wkr, Wo):
    own = (lax.axis_index("i") + 1) % N
    Wq_own = lax.dynamic_slice(Wq, (0, own * HB), (D, HB))
    Wqr_own = lax.dynamic_slice(Wqr, (0, own * RB), (D, RB))

    return pl.pallas_call(
        _body,
        out_shape=jax.ShapeDtypeStruct((B, S, D), F32),
        in_specs=[pl.BlockSpec(memory_space=pltpu.VMEM)] * 8,
        out_specs=pl.BlockSpec(memory_space=pltpu.VMEM),
        scratch_shapes=[
            pltpu.VMEM((B, S, 128), F32),
            pltpu.VMEM((N, B, S, 2 * HB), F32),
            pltpu.VMEM((N - 1, B, S, 2 * HB), F32),
            pltpu.VMEM((N, B, S, HB), F32),
            pltpu.VMEM((B, S, HB), F32),
            pltpu.VMEM((B, S, RB), F32),
            pltpu.VMEM((B, S, Dr), F32),
            pltpu.SemaphoreType.DMA((N - 1,)),
            pltpu.SemaphoreType.DMA((N - 1,)),
            pltpu.SemaphoreType.DMA((N - 1,)),
            pltpu.SemaphoreType.DMA((N - 1,)),
        ],
        compiler_params=pltpu.CompilerParams(collective_id=0),
    )(x, Wdkv, Wuk, Wuv, Wq_own, Wqr_own, Wkr, Wo)


# baseline (device time: 327261 ns/iter reference)
import jax
import jax.numpy as jnp
from jax import lax
from jax.experimental import pallas as pl
from jax.experimental.pallas import tpu as pltpu

N = 8
B, S, D = 2, 512, 2048
H, Dh, Dr = 16, 128, 32
HB = 256
RB = 64
F32 = jnp.float32
SCALE = (Dh + Dr) ** -0.5


def _body(x_ref, wdkv_ref, wuk_ref, wuv_ref, wq_ref, wqr_ref, wkr_ref,
          wo_ref, out_ref,
          c_ref, kvs_ref, rsbuf_ref, ob_ref, q_ref, qr_ref, kr_ref, wo_buf,
          rs_send_sems, rs_recv_sems, ag_send_sems, ag_recv_sems, wo_sem):

    def fetch_wo(blk):
        cp = pltpu.make_async_copy(
            wo_ref.at[pl.ds(blk * HB, HB), :], wo_buf, wo_sem)
        cp.start()
        cp.wait()
    my = lax.axis_index("i")
    left = (my + N - 1) % N
    right = (my + 1) % N
    own = right

    def kv_partial(t, dst_slot):
        for b in range(B):
            kvs_ref[dst_slot, b, :, 0:HB] = jnp.dot(
                c_ref[b], wuk_ref[:, t * HB:(t + 1) * HB],
                preferred_element_type=F32)
            kvs_ref[dst_slot, b, :, HB:2 * HB] = jnp.dot(
                c_ref[b], wuv_ref[:, t * HB:(t + 1) * HB],
                preferred_element_type=F32)

    for b in range(B):
        c_ref[b] = jnp.dot(x_ref[b], wdkv_ref[...], preferred_element_type=F32)
    kv_partial(0, 0)
    for b in range(B):
        q_ref[b] = jnp.dot(x_ref[b], wq_ref[...], preferred_element_type=F32)
        qr_ref[b] = jnp.dot(x_ref[b], wqr_ref[...], preferred_element_type=F32)
        kr_ref[b] = jnp.dot(x_ref[b], wkr_ref[...], preferred_element_type=F32)

    barrier_sem = pltpu.get_barrier_semaphore()
    for nbr in (left, right):
        pl.semaphore_signal(barrier_sem, inc=1, device_id=(nbr,),
                            device_id_type=pl.DeviceIdType.MESH)
    pl.semaphore_wait(barrier_sem, 2)

    for s in range(N - 1):
        rdma = pltpu.make_async_remote_copy(
            src_ref=kvs_ref.at[s % 2],
            dst_ref=rsbuf_ref.at[s],
            send_sem=rs_send_sems.at[s],
            recv_sem=rs_recv_sems.at[s],
            device_id=(right,),
            device_id_type=pl.DeviceIdType.MESH,
        )
        rdma.start()
        kv_partial(s + 1, (s + 1) % 2)
        rdma.wait()
        for b in range(B):
            kvs_ref[(s + 1) % 2, b] = kvs_ref[(s + 1) % 2, b] + rsbuf_ref[s, b]
    fin = (N - 1) % 2

    for b in range(B):
        for h in range(2):
            qh = q_ref[b, :, h * Dh:(h + 1) * Dh]
            kh = kvs_ref[fin, b, :, h * Dh:(h + 1) * Dh]
            vh = kvs_ref[fin, b, :, HB + h * Dh:HB + (h + 1) * Dh]
            qrh = qr_ref[b, :, h * Dr:(h + 1) * Dr]
            krb = kr_ref[b]
            sc = lax.dot_general(qh, kh, (((1,), (1,)), ((), ())),
                                 preferred_element_type=F32)
            sc = sc + lax.dot_general(qrh, krb, (((1,), (1,)), ((), ())),
                                      preferred_element_type=F32)
            sc = sc * SCALE
            m = jnp.max(sc, axis=-1, keepdims=True)
            p = jnp.exp(sc - m)
            p = p / jnp.sum(p, axis=-1, keepdims=True)
            ob_ref[own, b, :, h * Dh:(h + 1) * Dh] = jnp.dot(
                p, vh, preferred_element_type=F32)

    fetch_wo(own)
    for b in range(B):
        out_ref[b] = jnp.dot(ob_ref[own, b], wo_buf[...],
                             preferred_element_type=F32)

    for hop in range(N - 1):
        sblk = (my + 1 - hop) % N
        rdma = pltpu.make_async_remote_copy(
            src_ref=ob_ref.at[sblk],
            dst_ref=ob_ref.at[sblk],
            send_sem=ag_send_sems.at[hop],
            recv_sem=ag_recv_sems.at[hop],
            device_id=(right,),
            device_id_type=pl.DeviceIdType.MESH,
        )
        rdma.start()
        rdma.wait()
        rblk = (my - hop) % N
        fetch_wo(rblk)
        for b in range(B):
            out_ref[b] = out_ref[b] + jnp.dot(
                ob_ref[rblk, b], wo_buf[...],
                preferred_element_type=F32)


def kernel(x, Wdkv, Wuk, Wuv, Wq, Wqr, Wkr, Wo):
    my = lax.axis_index("i")
    own = (my + 1) % N
    Wq_own = lax.dynamic_slice(Wq, (0, own * HB), (D, HB))
    Wqr_own = lax.dynamic_slice(Wqr, (0, own * RB), (D, RB))
    perm = (my - jnp.arange(N)) % N
    Wuk_p = jnp.take(Wuk.reshape(128, N, HB), perm, axis=1).reshape(128, D)
    Wuv_p = jnp.take(Wuv.reshape(128, N, HB), perm, axis=1).reshape(128, D)

    return pl.pallas_call(
        _body,
        out_shape=jax.ShapeDtypeStruct((B, S, D), F32),
        in_specs=[pl.BlockSpec(memory_space=pltpu.VMEM)] * 7
        + [pl.BlockSpec(memory_space=pl.ANY)],
        out_specs=pl.BlockSpec(memory_space=pltpu.VMEM),
        scratch_shapes=[
            pltpu.VMEM((B, S, 128), F32),
            pltpu.VMEM((2, B, S, 2 * HB), F32),
            pltpu.VMEM((N - 1, B, S, 2 * HB), F32),
            pltpu.VMEM((N, B, S, HB), F32),
            pltpu.VMEM((B, S, HB), F32),
            pltpu.VMEM((B, S, RB), F32),
            pltpu.VMEM((B, S, Dr), F32),
            pltpu.VMEM((HB, D), F32),
            pltpu.SemaphoreType.DMA((N - 1,)),
            pltpu.SemaphoreType.DMA((N - 1,)),
            pltpu.SemaphoreType.DMA((N - 1,)),
            pltpu.SemaphoreType.DMA((N - 1,)),
            pltpu.SemaphoreType.DMA,
        ],
        compiler_params=pltpu.CompilerParams(
            collective_id=0, vmem_limit_bytes=100 * 1024 * 1024),
    )(x, Wdkv, Wuk_p, Wuv_p, Wq_own, Wqr_own, Wkr, Wo)


# device time: 225376 ns/iter; 1.4521x vs baseline; 1.4521x over previous
import jax
import jax.numpy as jnp
from jax import lax
from jax.experimental import pallas as pl
from jax.experimental.pallas import tpu as pltpu

N = 8
B, S, D = 2, 512, 2048
H, Dh, Dr = 16, 128, 32
HB = 256
RB = 64
F32 = jnp.float32
SCALE = (Dh + Dr) ** -0.5


def _body(x_ref, wdkv_ref, wuk_ref, wuv_ref, wq_ref, wqr_ref, wkr_ref,
          wo_ref, out_ref,
          c_ref, ks_ref, vs_ref, rbk_ref, rbv_ref, ob_ref,
          q_ref, qr_ref, kr_ref, wo_buf,
          kf_send, kf_recv, vb_send, vb_recv,
          ag_send, ag_recv, wo_sems):
    my = lax.axis_index("i")
    left = (my + N - 1) % N
    right = (my + 1) % N
    own = right

    def k_partial(t, slot):
        for b in range(B):
            ks_ref[slot, b] = jnp.dot(
                c_ref[b], wuk_ref[:, t * HB:(t + 1) * HB],
                preferred_element_type=F32)

    def v_partial(t, slot):
        for b in range(B):
            vs_ref[slot, b] = jnp.dot(
                c_ref[b], wuv_ref[:, t * HB:(t + 1) * HB],
                preferred_element_type=F32)

    def wo_fetch(t, slot):
        blk = (my + 1 - t) % N
        return pltpu.make_async_copy(
            wo_ref.at[pl.ds(blk * HB, HB), :], wo_buf.at[slot],
            wo_sems.at[slot])

    wo_fetch(0, 0).start()

    for b in range(B):
        c_ref[b] = jnp.dot(x_ref[b], wdkv_ref[...], preferred_element_type=F32)
    k_partial(0, 0)
    v_partial(0, 0)

    barrier_sem = pltpu.get_barrier_semaphore()
    for nbr in (left, right):
        pl.semaphore_signal(barrier_sem, inc=1, device_id=(nbr,),
                            device_id_type=pl.DeviceIdType.MESH)
    pl.semaphore_wait(barrier_sem, 2)

    for s in range(N - 1):
        rk = pltpu.make_async_remote_copy(
            src_ref=ks_ref.at[s % 2], dst_ref=rbk_ref.at[s],
            send_sem=kf_send.at[s], recv_sem=kf_recv.at[s],
            device_id=(right,), device_id_type=pl.DeviceIdType.MESH)
        rv = pltpu.make_async_remote_copy(
            src_ref=vs_ref.at[s % 2], dst_ref=rbv_ref.at[s],
            send_sem=vb_send.at[s], recv_sem=vb_recv.at[s],
            device_id=(left,), device_id_type=pl.DeviceIdType.MESH)
        rk.start()
        rv.start()
        if s == 0:
            for b in range(B):
                q_ref[b] = jnp.dot(x_ref[b], wq_ref[...],
                                   preferred_element_type=F32)
                qr_ref[b] = jnp.dot(x_ref[b], wqr_ref[...],
                                    preferred_element_type=F32)
                kr_ref[b] = jnp.dot(x_ref[b], wkr_ref[...],
                                    preferred_element_type=F32)
        k_partial(s + 1, (s + 1) % 2)
        v_partial(s + 1, (s + 1) % 2)
        rk.wait()
        rv.wait()
        for b in range(B):
            ks_ref[(s + 1) % 2, b] = ks_ref[(s + 1) % 2, b] + rbk_ref[s, b]
            vs_ref[(s + 1) % 2, b] = vs_ref[(s + 1) % 2, b] + rbv_ref[s, b]
    fin = (N - 1) % 2

    for b in range(B):
        for h in range(2):
            qh = q_ref[b, :, h * Dh:(h + 1) * Dh]
            kh = ks_ref[fin, b, :, h * Dh:(h + 1) * Dh]
            vh = vs_ref[fin, b, :, h * Dh:(h + 1) * Dh]
            qrh = qr_ref[b, :, h * Dr:(h + 1) * Dr]
            krb = kr_ref[b]
            sc = lax.dot_general(qh, kh, (((1,), (1,)), ((), ())),
                                 preferred_element_type=F32)
            sc = sc + lax.dot_general(qrh, krb, (((1,), (1,)), ((), ())),
                                      preferred_element_type=F32)
            sc = sc * SCALE
            m = jnp.max(sc, axis=-1, keepdims=True)
            p = jnp.exp(sc - m)
            p = p / jnp.sum(p, axis=-1, keepdims=True)
            ob_ref[own, b, :, h * Dh:(h + 1) * Dh] = jnp.dot(
                p, vh, preferred_element_type=F32)

    def proj(t, slot):
        blk = (my + 1 - t) % N
        cp = wo_fetch(t, slot)
        cp.wait()
        for b in range(B):
            o = jnp.dot(ob_ref[blk, b], wo_buf[slot],
                        preferred_element_type=F32)
            if t == 0:
                out_ref[b] = o
            else:
                out_ref[b] = out_ref[b] + o

    wo_fetch(1, 1).start()
    for hop in range(N - 1):
        sblk = (my + 1 - hop) % N
        rdma = pltpu.make_async_remote_copy(
            src_ref=ob_ref.at[sblk], dst_ref=ob_ref.at[sblk],
            send_sem=ag_send.at[hop], recv_sem=ag_recv.at[hop],
            device_id=(right,), device_id_type=pl.DeviceIdType.MESH)
        rdma.start()
        proj(hop, hop % 2)
        if hop + 2 < N:
            wo_fetch(hop + 2, hop % 2).start()
        rdma.wait()
    proj(N - 1, (N - 1) % 2)


def kernel(x, Wdkv, Wuk, Wuv, Wq, Wqr, Wkr, Wo):
    my = lax.axis_index("i")
    own = (my + 1) % N
    Wq_own = lax.dynamic_slice(Wq, (0, own * HB), (D, HB))
    Wqr_own = lax.dynamic_slice(Wqr, (0, own * RB), (D, RB))
    perm_k = (my - jnp.arange(N)) % N
    perm_v = (my + 2 + jnp.arange(N)) % N
    Wuk_p = jnp.take(Wuk.reshape(128, N, HB), perm_k, axis=1).reshape(128, D)
    Wuv_p = jnp.take(Wuv.reshape(128, N, HB), perm_v, axis=1).reshape(128, D)

    return pl.pallas_call(
        _body,
        out_shape=jax.ShapeDtypeStruct((B, S, D), F32),
        in_specs=[pl.BlockSpec(memory_space=pltpu.VMEM)] * 7
        + [pl.BlockSpec(memory_space=pl.ANY)],
        out_specs=pl.BlockSpec(memory_space=pltpu.VMEM),
        scratch_shapes=[
            pltpu.VMEM((B, S, 128), F32),
            pltpu.VMEM((2, B, S, HB), F32),
            pltpu.VMEM((2, B, S, HB), F32),
            pltpu.VMEM((N - 1, B, S, HB), F32),
            pltpu.VMEM((N - 1, B, S, HB), F32),
            pltpu.VMEM((N, B, S, HB), F32),
            pltpu.VMEM((B, S, HB), F32),
            pltpu.VMEM((B, S, RB), F32),
            pltpu.VMEM((B, S, Dr), F32),
            pltpu.VMEM((2, HB, D), F32),
            pltpu.SemaphoreType.DMA((N - 1,)),
            pltpu.SemaphoreType.DMA((N - 1,)),
            pltpu.SemaphoreType.DMA((N - 1,)),
            pltpu.SemaphoreType.DMA((N - 1,)),
            pltpu.SemaphoreType.DMA((N - 1,)),
            pltpu.SemaphoreType.DMA((N - 1,)),
            pltpu.SemaphoreType.DMA((2,)),
        ],
        compiler_params=pltpu.CompilerParams(
            collective_id=0, vmem_limit_bytes=100 * 1024 * 1024),
    )(x, Wdkv, Wuk_p, Wuv_p, Wq_own, Wqr_own, Wkr, Wo)


# device time: 186634 ns/iter; 1.7535x vs baseline; 1.2076x over previous
import jax
import jax.numpy as jnp
from jax import lax
from jax.experimental import pallas as pl
from jax.experimental.pallas import tpu as pltpu

N = 8
B, S, D = 2, 512, 2048
H, Dh, Dr = 16, 128, 32
HB = 256
RB = 64
F32 = jnp.float32
BF16 = jnp.bfloat16
SCALE = (Dh + Dr) ** -0.5


def _body(x_ref, wdkv_ref, wuk_ref, wuv_ref, wq_ref, wqr_ref, wkr_ref,
          wo_ref, out_ref,
          c_ref, ksend_ref, vsend_ref, rbk_ref, rbv_ref, kfin_ref, vfin_ref,
          ob_ref, q_ref, qr_ref, kr_ref, wo_buf,
          kf_send, kf_recv, vb_send, vb_recv,
          ag_send, ag_recv, wo_sems):
    my = lax.axis_index("i")
    left = (my + N - 1) % N
    right = (my + 1) % N
    own = right

    def k_partial(t):
        for b in range(B):
            kfin_ref[b] = jnp.dot(
                c_ref[b], wuk_ref[:, t * HB:(t + 1) * HB],
                preferred_element_type=F32)

    def v_partial(t):
        for b in range(B):
            vfin_ref[b] = jnp.dot(
                c_ref[b], wuv_ref[:, t * HB:(t + 1) * HB],
                preferred_element_type=F32)

    def wo_fetch(t, slot):
        blk = (my + 1 - t) % N
        return pltpu.make_async_copy(
            wo_ref.at[pl.ds(blk * HB, HB), :], wo_buf.at[slot],
            wo_sems.at[slot])

    wo_fetch(0, 0).start()

    for b in range(B):
        c_ref[b] = jnp.dot(x_ref[b], wdkv_ref[...], preferred_element_type=F32)
    k_partial(0)
    v_partial(0)
    for b in range(B):
        ksend_ref[0, b] = kfin_ref[b].astype(BF16)
        vsend_ref[0, b] = vfin_ref[b].astype(BF16)

    barrier_sem = pltpu.get_barrier_semaphore()
    for nbr in (left, right):
        pl.semaphore_signal(barrier_sem, inc=1, device_id=(nbr,),
                            device_id_type=pl.DeviceIdType.MESH)
    pl.semaphore_wait(barrier_sem, 2)

    for s in range(N - 1):
        rk = pltpu.make_async_remote_copy(
            src_ref=ksend_ref.at[s % 2], dst_ref=rbk_ref.at[s],
            send_sem=kf_send.at[s], recv_sem=kf_recv.at[s],
            device_id=(right,), device_id_type=pl.DeviceIdType.MESH)
        rv = pltpu.make_async_remote_copy(
            src_ref=vsend_ref.at[s % 2], dst_ref=rbv_ref.at[s],
            send_sem=vb_send.at[s], recv_sem=vb_recv.at[s],
            device_id=(left,), device_id_type=pl.DeviceIdType.MESH)
        rk.start()
        rv.start()
        if s == 0:
            for b in range(B):
                q_ref[b] = jnp.dot(x_ref[b], wq_ref[...],
                                   preferred_element_type=F32)
                qr_ref[b] = jnp.dot(x_ref[b], wqr_ref[...],
                                    preferred_element_type=F32)
                kr_ref[b] = jnp.dot(x_ref[b], wkr_ref[...],
                                    preferred_element_type=F32)
        k_partial(s + 1)
        v_partial(s + 1)
        rk.wait()
        rv.wait()
        last = s == N - 2
        for b in range(B):
            kacc = kfin_ref[b] + rbk_ref[s, b].astype(F32)
            vacc = vfin_ref[b] + rbv_ref[s, b].astype(F32)
            if last:
                kfin_ref[b] = kacc
                vfin_ref[b] = vacc
            else:
                ksend_ref[(s + 1) % 2, b] = kacc.astype(BF16)
                vsend_ref[(s + 1) % 2, b] = vacc.astype(BF16)

    for b in range(B):
        for h in range(2):
            qh = q_ref[b, :, h * Dh:(h + 1) * Dh]
            kh = kfin_ref[b, :, h * Dh:(h + 1) * Dh]
            vh = vfin_ref[b, :, h * Dh:(h + 1) * Dh]
            qrh = qr_ref[b, :, h * Dr:(h + 1) * Dr]
            krb = kr_ref[b]
            sc = lax.dot_general(qh, kh, (((1,), (1,)), ((), ())),
                                 preferred_element_type=F32)
            sc = sc + lax.dot_general(qrh, krb, (((1,), (1,)), ((), ())),
                                      preferred_element_type=F32)
            sc = sc * SCALE
            m = jnp.max(sc, axis=-1, keepdims=True)
            p = jnp.exp(sc - m)
            p = p / jnp.sum(p, axis=-1, keepdims=True)
            ob_ref[own, b, :, h * Dh:(h + 1) * Dh] = jnp.dot(
                p, vh, preferred_element_type=F32)

    def proj(t, slot):
        blk = (my + 1 - t) % N
        cp = wo_fetch(t, slot)
        cp.wait()
        for b in range(B):
            o = jnp.dot(ob_ref[blk, b], wo_buf[slot],
                        preferred_element_type=F32)
            if t == 0:
                out_ref[b] = o
            else:
                out_ref[b] = out_ref[b] + o

    wo_fetch(1, 1).start()
    for hop in range(N - 1):
        sblk = (my + 1 - hop) % N
        rdma = pltpu.make_async_remote_copy(
            src_ref=ob_ref.at[sblk], dst_ref=ob_ref.at[sblk],
            send_sem=ag_send.at[hop], recv_sem=ag_recv.at[hop],
            device_id=(right,), device_id_type=pl.DeviceIdType.MESH)
        rdma.start()
        proj(hop, hop % 2)
        if hop + 2 < N:
            wo_fetch(hop + 2, hop % 2).start()
        rdma.wait()
    proj(N - 1, (N - 1) % 2)


def kernel(x, Wdkv, Wuk, Wuv, Wq, Wqr, Wkr, Wo):
    my = lax.axis_index("i")
    own = (my + 1) % N
    Wq_own = lax.dynamic_slice(Wq, (0, own * HB), (D, HB))
    Wqr_own = lax.dynamic_slice(Wqr, (0, own * RB), (D, RB))
    perm_k = (my - jnp.arange(N)) % N
    perm_v = (my + 2 + jnp.arange(N)) % N
    Wuk_p = jnp.take(Wuk.reshape(128, N, HB), perm_k, axis=1).reshape(128, D)
    Wuv_p = jnp.take(Wuv.reshape(128, N, HB), perm_v, axis=1).reshape(128, D)

    return pl.pallas_call(
        _body,
        out_shape=jax.ShapeDtypeStruct((B, S, D), F32),
        in_specs=[pl.BlockSpec(memory_space=pltpu.VMEM)] * 7
        + [pl.BlockSpec(memory_space=pl.ANY)],
        out_specs=pl.BlockSpec(memory_space=pltpu.VMEM),
        scratch_shapes=[
            pltpu.VMEM((B, S, 128), F32),
            pltpu.VMEM((2, B, S, HB), BF16),
            pltpu.VMEM((2, B, S, HB), BF16),
            pltpu.VMEM((N - 1, B, S, HB), BF16),
            pltpu.VMEM((N - 1, B, S, HB), BF16),
            pltpu.VMEM((B, S, HB), F32),
            pltpu.VMEM((B, S, HB), F32),
            pltpu.VMEM((N, B, S, HB), F32),
            pltpu.VMEM((B, S, HB), F32),
            pltpu.VMEM((B, S, RB), F32),
            pltpu.VMEM((B, S, Dr), F32),
            pltpu.VMEM((2, HB, D), F32),
            pltpu.SemaphoreType.DMA((N - 1,)),
            pltpu.SemaphoreType.DMA((N - 1,)),
            pltpu.SemaphoreType.DMA((N - 1,)),
            pltpu.SemaphoreType.DMA((N - 1,)),
            pltpu.SemaphoreType.DMA((N - 1,)),
            pltpu.SemaphoreType.DMA((N - 1,)),
            pltpu.SemaphoreType.DMA((2,)),
        ],
        compiler_params=pltpu.CompilerParams(
            collective_id=0, vmem_limit_bytes=100 * 1024 * 1024),
    )(x, Wdkv, Wuk_p, Wuv_p, Wq_own, Wqr_own, Wkr, Wo)


# device time: 136893 ns/iter; 2.3906x vs baseline; 1.3634x over previous
import jax
import jax.numpy as jnp
from jax import lax
from jax.experimental import pallas as pl
from jax.experimental.pallas import tpu as pltpu

N = 8
B, S, D = 2, 512, 2048
H, Dh, Dr = 16, 128, 32
HB = 256
RB = 64
F32 = jnp.float32
BF16 = jnp.bfloat16
SCALE = (Dh + Dr) ** -0.5


def _body(x_ref, wdkv_ref, wuk_ref, wuv_ref, wq_ref, wqr_ref, wkr_ref,
          wo_ref, out_ref,
          c_ref, ksend_ref, vsend_ref, rbk_ref, rbv_ref, kfin_ref, vfin_ref,
          obf_ref, obb_ref, q_ref, qr_ref, kr_ref, wof_buf, wob_buf,
          kf_send, kf_recv, vb_send, vb_recv,
          agf_send, agf_recv, agb_send, agb_recv, wof_sems, wob_sems):
    my = lax.axis_index("i")
    left = (my + N - 1) % N
    right = (my + 1) % N
    own = right

    def k_partial(t):
        for b in range(B):
            kfin_ref[b] = jnp.dot(
                c_ref[b], wuk_ref[:, t * HB:(t + 1) * HB],
                preferred_element_type=F32)

    def v_partial(t):
        for b in range(B):
            vfin_ref[b] = jnp.dot(
                c_ref[b], wuv_ref[:, t * HB:(t + 1) * HB],
                preferred_element_type=F32)

    def wof_fetch(t, slot):
        blk = (my + 1 - t) % N
        return pltpu.make_async_copy(
            wo_ref.at[pl.ds(blk * HB, Dh), :], wof_buf.at[slot],
            wof_sems.at[slot])

    def wob_fetch(t, slot):
        blk = (my + 1 + t) % N
        return pltpu.make_async_copy(
            wo_ref.at[pl.ds(blk * HB + Dh, Dh), :], wob_buf.at[slot],
            wob_sems.at[slot])

    wof_fetch(0, 0).start()
    wob_fetch(0, 0).start()
    wof_fetch(1, 1).start()
    wob_fetch(1, 1).start()

    for b in range(B):
        c_ref[b] = jnp.dot(x_ref[b], wdkv_ref[...], preferred_element_type=F32)
    k_partial(0)
    v_partial(0)
    for b in range(B):
        ksend_ref[0, b] = kfin_ref[b].astype(BF16)
        vsend_ref[0, b] = vfin_ref[b].astype(BF16)

    barrier_sem = pltpu.get_barrier_semaphore()
    for nbr in (left, right):
        pl.semaphore_signal(barrier_sem, inc=1, device_id=(nbr,),
                            device_id_type=pl.DeviceIdType.MESH)
    pl.semaphore_wait(barrier_sem, 2)

    for s in range(N - 1):
        rk = pltpu.make_async_remote_copy(
            src_ref=ksend_ref.at[s % 2], dst_ref=rbk_ref.at[s],
            send_sem=kf_send.at[s], recv_sem=kf_recv.at[s],
            device_id=(right,), device_id_type=pl.DeviceIdType.MESH)
        rv = pltpu.make_async_remote_copy(
            src_ref=vsend_ref.at[s % 2], dst_ref=rbv_ref.at[s],
            send_sem=vb_send.at[s], recv_sem=vb_recv.at[s],
            device_id=(left,), device_id_type=pl.DeviceIdType.MESH)
        rk.start()
        rv.start()
        if s == 0:
            for b in range(B):
                q_ref[b] = jnp.dot(x_ref[b], wq_ref[...],
                                   preferred_element_type=F32)
                qr_ref[b] = jnp.dot(x_ref[b], wqr_ref[...],
                                    preferred_element_type=F32)
                kr_ref[b] = jnp.dot(x_ref[b], wkr_ref[...],
                                    preferred_element_type=F32)
        k_partial(s + 1)
        v_partial(s + 1)
        rk.wait()
        rv.wait()
        last = s == N - 2
        for b in range(B):
            kacc = kfin_ref[b] + rbk_ref[s, b].astype(F32)
            vacc = vfin_ref[b] + rbv_ref[s, b].astype(F32)
            if last:
                kfin_ref[b] = kacc
                vfin_ref[b] = vacc
            else:
                ksend_ref[(s + 1) % 2, b] = kacc.astype(BF16)
                vsend_ref[(s + 1) % 2, b] = vacc.astype(BF16)

    def attn_head(h, dst_ref):
        for b in range(B):
            qh = q_ref[b, :, h * Dh:(h + 1) * Dh]
            kh = kfin_ref[b, :, h * Dh:(h + 1) * Dh]
            vh = vfin_ref[b, :, h * Dh:(h + 1) * Dh]
            qrh = qr_ref[b, :, h * Dr:(h + 1) * Dr]
            krb = kr_ref[b]
            sc = lax.dot_general(qh, kh, (((1,), (1,)), ((), ())),
                                 preferred_element_type=F32)
            sc = sc + lax.dot_general(qrh, krb, (((1,), (1,)), ((), ())),
                                      preferred_element_type=F32)
            sc = sc * SCALE
            m = jnp.max(sc, axis=-1, keepdims=True)
            p = jnp.exp(sc - m)
            p = p / jnp.sum(p, axis=-1, keepdims=True)
            dst_ref[own, b] = jnp.dot(
                p, vh, preferred_element_type=F32).astype(BF16)

    def fwd_rdma(hop):
        sblk = (my + 1 - hop) % N
        return pltpu.make_async_remote_copy(
            src_ref=obf_ref.at[sblk], dst_ref=obf_ref.at[sblk],
            send_sem=agf_send.at[hop], recv_sem=agf_recv.at[hop],
            device_id=(right,), device_id_type=pl.DeviceIdType.MESH)

    def bwd_rdma(hop):
        sblk = (my + 1 + hop) % N
        return pltpu.make_async_remote_copy(
            src_ref=obb_ref.at[sblk], dst_ref=obb_ref.at[sblk],
            send_sem=agb_send.at[hop], recv_sem=agb_recv.at[hop],
            device_id=(left,), device_id_type=pl.DeviceIdType.MESH)

    attn_head(0, obf_ref)
    fwd_rdma(0).start()
    attn_head(1, obb_ref)

    def proj_f(t, slot):
        blk = (my + 1 - t) % N
        wof_fetch(t, slot).wait()
        for b in range(B):
            o = jnp.dot(obf_ref[blk, b], wof_buf[slot],
                        preferred_element_type=F32)
            if t == 0:
                out_ref[b] = o
            else:
                out_ref[b] = out_ref[b] + o

    def proj_b(t, slot):
        blk = (my + 1 + t) % N
        wob_fetch(t, slot).wait()
        for b in range(B):
            out_ref[b] = out_ref[b] + jnp.dot(
                obb_ref[blk, b], wob_buf[slot], preferred_element_type=F32)

    for hop in range(N - 1):
        rf = fwd_rdma(hop)
        rb = bwd_rdma(hop)
        if hop > 0:
            rf.start()
        rb.start()
        proj_f(hop, hop % 2)
        proj_b(hop, hop % 2)
        if hop + 2 < N:
            wof_fetch(hop + 2, hop % 2).start()
            wob_fetch(hop + 2, hop % 2).start()
        rf.wait()
        rb.wait()
    proj_f(N - 1, (N - 1) % 2)
    proj_b(N - 1, (N - 1) % 2)


def kernel(x, Wdkv, Wuk, Wuv, Wq, Wqr, Wkr, Wo):
    my = lax.axis_index("i")
    own = (my + 1) % N
    Wq_own = lax.dynamic_slice(Wq, (0, own * HB), (D, HB))
    Wqr_own = lax.dynamic_slice(Wqr, (0, own * RB), (D, RB))
    perm_k = (my - jnp.arange(N)) % N
    perm_v = (my + 2 + jnp.arange(N)) % N
    Wuk_p = jnp.take(Wuk.reshape(128, N, HB), perm_k, axis=1).reshape(128, D)
    Wuv_p = jnp.take(Wuv.reshape(128, N, HB), perm_v, axis=1).reshape(128, D)
    Wo_bf = Wo.astype(BF16)

    return pl.pallas_call(
        _body,
        out_shape=jax.ShapeDtypeStruct((B, S, D), F32),
        in_specs=[pl.BlockSpec(memory_space=pltpu.VMEM)] * 7
        + [pl.BlockSpec(memory_space=pl.ANY)],
        out_specs=pl.BlockSpec(memory_space=pltpu.VMEM),
        scratch_shapes=[
            pltpu.VMEM((B, S, 128), F32),
            pltpu.VMEM((2, B, S, HB), BF16),
            pltpu.VMEM((2, B, S, HB), BF16),
            pltpu.VMEM((N - 1, B, S, HB), BF16),
            pltpu.VMEM((N - 1, B, S, HB), BF16),
            pltpu.VMEM((B, S, HB), F32),
            pltpu.VMEM((B, S, HB), F32),
            pltpu.VMEM((N, B, S, Dh), BF16),
            pltpu.VMEM((N, B, S, Dh), BF16),
            pltpu.VMEM((B, S, HB), F32),
            pltpu.VMEM((B, S, RB), F32),
            pltpu.VMEM((B, S, Dr), F32),
            pltpu.VMEM((2, Dh, D), BF16),
            pltpu.VMEM((2, Dh, D), BF16),
            pltpu.SemaphoreType.DMA((N - 1,)),
            pltpu.SemaphoreType.DMA((N - 1,)),
            pltpu.SemaphoreType.DMA((N - 1,)),
            pltpu.SemaphoreType.DMA((N - 1,)),
            pltpu.SemaphoreType.DMA((N - 1,)),
            pltpu.SemaphoreType.DMA((N - 1,)),
            pltpu.SemaphoreType.DMA((N - 1,)),
            pltpu.SemaphoreType.DMA((N - 1,)),
            pltpu.SemaphoreType.DMA((2,)),
            pltpu.SemaphoreType.DMA((2,)),
        ],
        compiler_params=pltpu.CompilerParams(
            collective_id=0, vmem_limit_bytes=100 * 1024 * 1024),
    )(x, Wdkv, Wuk_p, Wuv_p, Wq_own, Wqr_own, Wkr, Wo_bf)
